# baseline (speedup 1.0000x reference)
import functools

import jax
import jax.numpy as jnp
from jax import lax
from jax.experimental import pallas as pl
from jax.experimental.pallas import tpu as pltpu
from jax.experimental.pallas import tpu_sc as plsc

B = 16384
F = 16
NC, NS = 2, 16
NW = NC * NS
BPW = B // NW
CHUNK = 128
NCHUNK = BPW // CHUNK


def _sc_gather_body(uid_hbm, iid_hbm, p_hbm, q_hbm, u_hbm, v_hbm,
                    pmf_hbm, qmf_hbm, pml_hbm, qml_hbm,
                    uidx, iidx, bp_v, bq_v, bu_v, bv_v, sem):
    wid = lax.axis_index("s") * NC + lax.axis_index("c")
    idx_row = wid * NCHUNK
    pltpu.sync_copy(uid_hbm.at[pl.ds(idx_row, NCHUNK)], uidx)
    pltpu.sync_copy(iid_hbm.at[pl.ds(idx_row, NCHUNK)], iidx)
    copies = []
    for c in range(NCHUNK):
        dst = pl.ds(c * CHUNK, CHUNK)
        copies.append(pltpu.async_copy(p_hbm.at[uidx.at[c]], bp_v.at[dst], sem))
        copies.append(pltpu.async_copy(q_hbm.at[iidx.at[c]], bq_v.at[dst], sem))
        copies.append(pltpu.async_copy(u_hbm.at[uidx.at[c]], bu_v.at[dst], sem))
        copies.append(pltpu.async_copy(v_hbm.at[iidx.at[c]], bv_v.at[dst], sem))
    for cp in copies:
        cp.wait()
    out_row = wid * BPW
    pltpu.sync_copy(bp_v, pmf_hbm.at[pl.ds(out_row, BPW)])
    pltpu.sync_copy(bq_v, qmf_hbm.at[pl.ds(out_row, BPW)])
    pltpu.sync_copy(bu_v, pml_hbm.at[pl.ds(out_row, BPW)])
    pltpu.sync_copy(bv_v, qml_hbm.at[pl.ds(out_row, BPW)])


@functools.lru_cache(maxsize=1)
def _sc_gather_call():
    mesh = plsc.VectorSubcoreMesh(core_axis_name="c", subcore_axis_name="s")
    return pl.kernel(
        _sc_gather_body,
        out_type=[jax.ShapeDtypeStruct((B, F), jnp.float32)] * 4,
        mesh=mesh,
        scratch_types=[
            pltpu.VMEM((NCHUNK, CHUNK), jnp.int32),
            pltpu.VMEM((NCHUNK, CHUNK), jnp.int32),
            pltpu.VMEM((BPW, F), jnp.float32),
            pltpu.VMEM((BPW, F), jnp.float32),
            pltpu.VMEM((BPW, F), jnp.float32),
            pltpu.VMEM((BPW, F), jnp.float32),
            pltpu.SemaphoreType.DMA,
        ],
        compiler_params=pltpu.CompilerParams(use_tc_tiling_on_sc=False),
    )


ROWS_PER_TC_BLOCK = 2048


def _tc_mlp_body(pmf_ref, qmf_ref, pml_ref, qml_ref,
                 w0_ref, b0_ref, w1_ref, b1_ref, w2_ref, b2_ref,
                 wpt_ref, bp_ref, out_ref):
    gmf = pmf_ref[:] * qmf_ref[:]
    x = jnp.concatenate([pml_ref[:], qml_ref[:]], axis=1)
    h = jnp.dot(x, w0_ref[:], preferred_element_type=jnp.float32) + b0_ref[:]
    h = jnp.maximum(h, 0.0)
    h = jnp.dot(h, w1_ref[:], preferred_element_type=jnp.float32) + b1_ref[:]
    h = jnp.maximum(h, 0.0)
    h = jnp.dot(h, w2_ref[:], preferred_element_type=jnp.float32) + b2_ref[:]
    h = jnp.maximum(h, 0.0)
    con = jnp.concatenate([gmf, h], axis=1)
    z = jnp.sum(con * wpt_ref[:], axis=1, keepdims=True) + bp_ref[:]
    out_ref[:] = jax.nn.sigmoid(z)


@functools.lru_cache(maxsize=1)
def _tc_mlp_call():
    nblk = B // ROWS_PER_TC_BLOCK
    row_spec = pl.BlockSpec((ROWS_PER_TC_BLOCK, F), lambda i: (i, 0))
    full = lambda shape: pl.BlockSpec(shape, lambda i: (0,) * len(shape))
    return pl.pallas_call(
        _tc_mlp_body,
        grid=(nblk,),
        in_specs=[
            row_spec, row_spec, row_spec, row_spec,
            full((2 * F, 64)), full((1, 64)),
            full((64, 32)), full((1, 32)),
            full((32, F)), full((1, F)),
            full((1, 2 * F)), full((1, 1)),
        ],
        out_specs=pl.BlockSpec((ROWS_PER_TC_BLOCK, 1), lambda i: (i, 0)),
        out_shape=jax.ShapeDtypeStruct((B, 1), jnp.float32),
    )


def kernel(user_id, item_id, P, Q, U_emb, V_emb, W0, b0, W1, b1, W2, b2, Wp, bp):
    uid = user_id.astype(jnp.int32).reshape(NW * NCHUNK, CHUNK)
    iid = item_id.astype(jnp.int32).reshape(NW * NCHUNK, CHUNK)
    pmf, qmf, pml, qml = _sc_gather_call()(uid, iid, P, Q, U_emb, V_emb)
    wpt = Wp.reshape(1, 2 * F)
    return _tc_mlp_call()(
        pmf, qmf, pml, qml,
        W0, b0.reshape(1, 64), W1, b1.reshape(1, 32), W2, b2.reshape(1, F),
        wpt, bp.reshape(1, 1),
    )


# speedup vs baseline: 2.4758x; 2.4758x over previous
import functools

import jax
import jax.numpy as jnp
from jax import lax
from jax.experimental import pallas as pl
from jax.experimental.pallas import tpu as pltpu
from jax.experimental.pallas import tpu_sc as plsc

B = 16384
F = 16
NC, NS = 2, 16
NW = NC * NS
BPW = B // NW
IDXW = 128
IDXROWS = BPW // IDXW
CG = 16
CPR = IDXW // CG
NCHUNK = BPW // CG
LINES = 125000


def _sc_gather_body(uid_hbm, iid_hbm, p_hbm, q_hbm, u_hbm, v_hbm,
                    gmf_hbm, xcat_hbm,
                    uid_s, iid_s,
                    bp_v, bq_v, gmf_v, xcat_v, sem):
    wid = lax.axis_index("s") * NC + lax.axis_index("c")
    idx_row = wid * IDXROWS
    pltpu.sync_copy(uid_hbm.at[pl.ds(idx_row, IDXROWS)], uid_s)
    pltpu.sync_copy(iid_hbm.at[pl.ds(idx_row, IDXROWS)], iid_s)

    def chunk_body(c, carry):
        r = c // CPR
        col = (c % CPR) * CG
        uvec = uid_s[r, pl.ds(col, CG)]
        ivec = iid_s[r, pl.ds(col, CG)]
        luv = lax.shift_right_logical(uvec, 3)
        juv = lax.bitwise_and(uvec, 7)
        liv = lax.shift_right_logical(ivec, 3)
        jiv = lax.bitwise_and(ivec, 7)
        cps = []
        for i in range(CG):
            lu, ju = luv[i], juv[i]
            li, ji = liv[i], jiv[i]
            cps.append(pltpu.async_copy(p_hbm.at[lu, ju], bp_v.at[i], sem))
            cps.append(pltpu.async_copy(q_hbm.at[li, ji], bq_v.at[i], sem))
            cps.append(pltpu.async_copy(
                u_hbm.at[lu, ju], xcat_v.at[i, pl.ds(0, F)], sem))
            cps.append(pltpu.async_copy(
                v_hbm.at[li, ji], xcat_v.at[i, pl.ds(F, F)], sem))
        for cp in cps:
            cp.wait()
        for i in range(CG):
            gmf_v[i, :] = bp_v[i, :] * bq_v[i, :]
        out = wid * BPW + c * CG
        pltpu.sync_copy(gmf_v, gmf_hbm.at[pl.ds(out, CG)])
        pltpu.sync_copy(xcat_v, xcat_hbm.at[pl.ds(out, CG)])
        return carry

    lax.fori_loop(0, NCHUNK, chunk_body, 0)


@functools.lru_cache(maxsize=1)
def _sc_gather_call():
    mesh = plsc.VectorSubcoreMesh(core_axis_name="c", subcore_axis_name="s")
    return pl.kernel(
        _sc_gather_body,
        out_type=[jax.ShapeDtypeStruct((B, F), jnp.float32),
                  jax.ShapeDtypeStruct((B, 2 * F), jnp.float32)],
        mesh=mesh,
        scratch_types=[
            pltpu.VMEM((IDXROWS, IDXW), jnp.int32),
            pltpu.VMEM((IDXROWS, IDXW), jnp.int32),
            pltpu.VMEM((CG, F), jnp.float32),
            pltpu.VMEM((CG, F), jnp.float32),
            pltpu.VMEM((CG, F), jnp.float32),
            pltpu.VMEM((CG, 2 * F), jnp.float32),
            pltpu.SemaphoreType.DMA,
        ],
        compiler_params=pltpu.CompilerParams(use_tc_tiling_on_sc=True),
    )


ROWS_PER_TC_BLOCK = 2048


def _tc_mlp_body(gmf_ref, xcat_ref,
                 w0_ref, b0_ref, w1_ref, b1_ref, w2_ref, b2_ref,
                 wpt_ref, bp_ref, out_ref):
    x = xcat_ref[:]
    h = jnp.dot(x, w0_ref[:], preferred_element_type=jnp.float32) + b0_ref[:]
    h = jnp.maximum(h, 0.0)
    h = jnp.dot(h, w1_ref[:], preferred_element_type=jnp.float32) + b1_ref[:]
    h = jnp.maximum(h, 0.0)
    h = jnp.dot(h, w2_ref[:], preferred_element_type=jnp.float32) + b2_ref[:]
    h = jnp.maximum(h, 0.0)
    con = jnp.concatenate([gmf_ref[:], h], axis=1)
    z = jnp.sum(con * wpt_ref[:], axis=1, keepdims=True) + bp_ref[:]
    out_ref[:] = jax.nn.sigmoid(z)


@functools.lru_cache(maxsize=1)
def _tc_mlp_call():
    nblk = B // ROWS_PER_TC_BLOCK
    full = lambda shape: pl.BlockSpec(shape, lambda i: (0,) * len(shape))
    return pl.pallas_call(
        _tc_mlp_body,
        grid=(nblk,),
        in_specs=[
            pl.BlockSpec((ROWS_PER_TC_BLOCK, F), lambda i: (i, 0)),
            pl.BlockSpec((ROWS_PER_TC_BLOCK, 2 * F), lambda i: (i, 0)),
            full((2 * F, 64)), full((1, 64)),
            full((64, 32)), full((1, 32)),
            full((32, F)), full((1, F)),
            full((1, 2 * F)), full((1, 1)),
        ],
        out_specs=pl.BlockSpec((ROWS_PER_TC_BLOCK, 1), lambda i: (i, 0)),
        out_shape=jax.ShapeDtypeStruct((B, 1), jnp.float32),
    )


def kernel(user_id, item_id, P, Q, U_emb, V_emb, W0, b0, W1, b1, W2, b2, Wp, bp):
    uid = user_id.astype(jnp.int32).reshape(B // IDXW, IDXW)
    iid = item_id.astype(jnp.int32).reshape(B // IDXW, IDXW)
    p3 = P.reshape(LINES, 8, F)
    q3 = Q.reshape(LINES, 8, F)
    u3 = U_emb.reshape(LINES, 8, F)
    v3 = V_emb.reshape(LINES, 8, F)
    gmf, xcat = _sc_gather_call()(uid, iid, p3, q3, u3, v3)
    wpt = Wp.reshape(1, 2 * F)
    return _tc_mlp_call()(
        gmf, xcat,
        W0, b0.reshape(1, 64), W1, b1.reshape(1, 32), W2, b2.reshape(1, F),
        wpt, bp.reshape(1, 1),
    )


# speedup vs baseline: 5.2393x; 2.1162x over previous
import functools

import jax
import jax.numpy as jnp
from jax import lax
from jax.experimental import pallas as pl
from jax.experimental.pallas import tpu as pltpu
from jax.experimental.pallas import tpu_sc as plsc

B = 16384
F = 16
N_ROWS = 1000000
NC, NS = 2, 16
NW = NC * NS
BPW = B // NW
IDXW = 128
IDXROWS = BPW // IDXW
GG = 16
NGRP = BPW // GG
HG = 8


def _sc_gather_body(uid_hbm, iid_hbm, p_hbm, q_hbm, u_hbm, v_hbm,
                    gmf_hbm, xcat_hbm,
                    uid_s, iid_s, sbuf, gmf_v, xcat_v, sem):
    wid = lax.axis_index("s") * NC + lax.axis_index("c")
    idx_row = wid * IDXROWS
    pltpu.sync_copy(uid_hbm.at[pl.ds(idx_row, IDXROWS)], uid_s)
    pltpu.sync_copy(iid_hbm.at[pl.ds(idx_row, IDXROWS)], iid_s)
    lanes = lax.iota(jnp.int32, 16)
    g_idx = lax.shift_right_logical(lanes, 3)
    f_idx = lax.bitwise_and(lanes, 7)

    def group_body(c, carry):
        r = c // (IDXW // GG)
        col = (c % (IDXW // GG)) * GG
        uvec = uid_s[r, pl.ds(col, GG)]
        ivec = iid_s[r, pl.ds(col, GG)]
        ubase = lax.bitwise_and(uvec, ~127)
        ibase = lax.bitwise_and(ivec, ~127)
        ulane = lax.bitwise_and(uvec, 127)
        ilane = lax.bitwise_and(ivec, 127)
        for h in range(GG // HG):
            cps = []
            for i in range(HG):
                j = h * HG + i
                ub = pl.multiple_of(ubase[j], 128)
                ib = pl.multiple_of(ibase[j], 128)
                cps.append(pltpu.async_copy(
                    p_hbm.at[:, :, pl.ds(ub, 128)], sbuf.at[i, 0], sem))
                cps.append(pltpu.async_copy(
                    q_hbm.at[:, :, pl.ds(ib, 128)], sbuf.at[i, 1], sem))
                cps.append(pltpu.async_copy(
                    u_hbm.at[:, :, pl.ds(ub, 128)], sbuf.at[i, 2], sem))
                cps.append(pltpu.async_copy(
                    v_hbm.at[:, :, pl.ds(ib, 128)], sbuf.at[i, 3], sem))
            for cp in cps:
                cp.wait()
            for i in range(HG):
                j = h * HG + i
                ul = jnp.full((16,), ulane[j], jnp.int32)
                il = jnp.full((16,), ilane[j], jnp.int32)
                prow = plsc.load_gather(sbuf.at[i, 0], [g_idx, f_idx, ul])
                qrow = plsc.load_gather(sbuf.at[i, 1], [g_idx, f_idx, il])
                urow = plsc.load_gather(sbuf.at[i, 2], [g_idx, f_idx, ul])
                vrow = plsc.load_gather(sbuf.at[i, 3], [g_idx, f_idx, il])
                gmf_v[j, :] = prow * qrow
                xcat_v[j, pl.ds(0, F)] = urow
                xcat_v[j, pl.ds(F, F)] = vrow
        out = wid * BPW + c * GG
        pltpu.sync_copy(gmf_v, gmf_hbm.at[pl.ds(out, GG)])
        pltpu.sync_copy(xcat_v, xcat_hbm.at[pl.ds(out, GG)])
        return carry

    lax.fori_loop(0, NGRP, group_body, 0)


@functools.lru_cache(maxsize=1)
def _sc_gather_call():
    mesh = plsc.VectorSubcoreMesh(core_axis_name="c", subcore_axis_name="s")
    return pl.kernel(
        _sc_gather_body,
        out_type=[jax.ShapeDtypeStruct((B, F), jnp.float32),
                  jax.ShapeDtypeStruct((B, 2 * F), jnp.float32)],
        mesh=mesh,
        scratch_types=[
            pltpu.VMEM((IDXROWS, IDXW), jnp.int32),
            pltpu.VMEM((IDXROWS, IDXW), jnp.int32),
            pltpu.VMEM((HG, 4, 2, 8, 128), jnp.float32),
            pltpu.VMEM((GG, F), jnp.float32),
            pltpu.VMEM((GG, 2 * F), jnp.float32),
            pltpu.SemaphoreType.DMA,
        ],
        compiler_params=pltpu.CompilerParams(
            use_tc_tiling_on_sc=True, needs_layout_passes=False),
    )


ROWS_PER_TC_BLOCK = 2048


def _tc_mlp_body(gmf_ref, xcat_ref,
                 w0_ref, b0_ref, w1_ref, b1_ref, w2_ref, b2_ref,
                 wpt_ref, bp_ref, out_ref):
    x = xcat_ref[:]
    h = jnp.dot(x, w0_ref[:], preferred_element_type=jnp.float32) + b0_ref[:]
    h = jnp.maximum(h, 0.0)
    h = jnp.dot(h, w1_ref[:], preferred_element_type=jnp.float32) + b1_ref[:]
    h = jnp.maximum(h, 0.0)
    h = jnp.dot(h, w2_ref[:], preferred_element_type=jnp.float32) + b2_ref[:]
    h = jnp.maximum(h, 0.0)
    con = jnp.concatenate([gmf_ref[:], h], axis=1)
    z = jnp.sum(con * wpt_ref[:], axis=1, keepdims=True) + bp_ref[:]
    out_ref[:] = jax.nn.sigmoid(z)


@functools.lru_cache(maxsize=1)
def _tc_mlp_call():
    nblk = B // ROWS_PER_TC_BLOCK
    full = lambda shape: pl.BlockSpec(shape, lambda i: (0,) * len(shape))
    return pl.pallas_call(
        _tc_mlp_body,
        grid=(nblk,),
        in_specs=[
            pl.BlockSpec((ROWS_PER_TC_BLOCK, F), lambda i: (i, 0)),
            pl.BlockSpec((ROWS_PER_TC_BLOCK, 2 * F), lambda i: (i, 0)),
            full((2 * F, 64)), full((1, 64)),
            full((64, 32)), full((1, 32)),
            full((32, F)), full((1, F)),
            full((1, 2 * F)), full((1, 1)),
        ],
        out_specs=pl.BlockSpec((ROWS_PER_TC_BLOCK, 1), lambda i: (i, 0)),
        out_shape=jax.ShapeDtypeStruct((B, 1), jnp.float32),
    )


def kernel(user_id, item_id, P, Q, U_emb, V_emb, W0, b0, W1, b1, W2, b2, Wp, bp):
    uid = user_id.astype(jnp.int32).reshape(B // IDXW, IDXW)
    iid = item_id.astype(jnp.int32).reshape(B // IDXW, IDXW)
    p3 = P.T.reshape(2, 8, N_ROWS)
    q3 = Q.T.reshape(2, 8, N_ROWS)
    u3 = U_emb.T.reshape(2, 8, N_ROWS)
    v3 = V_emb.T.reshape(2, 8, N_ROWS)
    gmf, xcat = _sc_gather_call()(uid, iid, p3, q3, u3, v3)
    wpt = Wp.reshape(1, 2 * F)
    return _tc_mlp_call()(
        gmf, xcat,
        W0, b0.reshape(1, 64), W1, b1.reshape(1, 32), W2, b2.reshape(1, F),
        wpt, bp.reshape(1, 1),
    )


# speedup vs baseline: 5.2682x; 1.0055x over previous
import functools

import jax
import jax.numpy as jnp
from jax import lax
from jax.experimental import pallas as pl
from jax.experimental.pallas import tpu as pltpu
from jax.experimental.pallas import tpu_sc as plsc

B = 16384
F = 16
N_ROWS = 1000000
NC, NS = 2, 16
NW = NC * NS
BPW = B // NW
IDXW = 128
IDXROWS = BPW // IDXW
GG = 16
NGRP = BPW // GG
HG = 4


def _sc_gather_body(uid_hbm, iid_hbm, p_hbm, q_hbm, u_hbm, v_hbm,
                    gmf_hbm, xcat_hbm,
                    uid_s, iid_s, sbuf, gmf_v, xcat_v, sem0, sem1):
    wid = lax.axis_index("s") * NC + lax.axis_index("c")
    idx_row = wid * IDXROWS
    pltpu.sync_copy(uid_hbm.at[pl.ds(idx_row, IDXROWS)], uid_s)
    pltpu.sync_copy(iid_hbm.at[pl.ds(idx_row, IDXROWS)], iid_s)
    lanes = lax.iota(jnp.int32, 16)
    g_idx = lax.shift_right_logical(lanes, 3)
    f_idx = lax.bitwise_and(lanes, 7)
    sems = (sem0, sem1)
    NH = GG // HG

    def idx_vecs(c):
        r = c // (IDXW // GG)
        col = (c % (IDXW // GG)) * GG
        uvec = uid_s[r, pl.ds(col, GG)]
        ivec = iid_s[r, pl.ds(col, GG)]
        return uvec, ivec

    def fire(uvec, ivec, h, slot):
        sem = sems[slot]
        for i in range(HG):
            j = h * HG + i
            ub = pl.multiple_of(lax.bitwise_and(uvec[j], ~127), 128)
            ib = pl.multiple_of(lax.bitwise_and(ivec[j], ~127), 128)
            pltpu.async_copy(p_hbm.at[:, :, pl.ds(ub, 128)],
                             sbuf.at[slot, i, 0], sem)
            pltpu.async_copy(q_hbm.at[:, :, pl.ds(ib, 128)],
                             sbuf.at[slot, i, 1], sem)
            pltpu.async_copy(u_hbm.at[:, :, pl.ds(ub, 128)],
                             sbuf.at[slot, i, 2], sem)
            pltpu.async_copy(v_hbm.at[:, :, pl.ds(ib, 128)],
                             sbuf.at[slot, i, 3], sem)

    def drain(slot):
        sem = sems[slot]
        for i in range(HG):
            for t in range(4):
                pltpu.make_async_copy(p_hbm.at[:, :, pl.ds(0, 128)],
                                      sbuf.at[slot, i, t], sem).wait()

    def extract(uvec, ivec, h, slot):
        for i in range(HG):
            j = h * HG + i
            ul = jnp.full((16,), lax.bitwise_and(uvec[j], 127), jnp.int32)
            il = jnp.full((16,), lax.bitwise_and(ivec[j], 127), jnp.int32)
            prow = plsc.load_gather(sbuf.at[slot, i, 0], [g_idx, f_idx, ul])
            qrow = plsc.load_gather(sbuf.at[slot, i, 1], [g_idx, f_idx, il])
            urow = plsc.load_gather(sbuf.at[slot, i, 2], [g_idx, f_idx, ul])
            vrow = plsc.load_gather(sbuf.at[slot, i, 3], [g_idx, f_idx, il])
            gmf_v[j, :] = prow * qrow
            xcat_v[j, pl.ds(0, F)] = urow
            xcat_v[j, pl.ds(F, F)] = vrow

    uvec0, ivec0 = idx_vecs(0)
    fire(uvec0, ivec0, 0, 0)

    def group_body(c, carry):
        uvec, ivec = idx_vecs(c)
        cn = jnp.minimum(c + 1, NGRP - 1)
        uvecn, ivecn = idx_vecs(cn)
        for h in range(NH):
            slot = h % 2
            nslot = (h + 1) % 2
            if h + 1 < NH:
                fire(uvec, ivec, h + 1, nslot)
            drain(slot)
            extract(uvec, ivec, h, slot)
            if h + 1 == NH:
                fire(uvecn, ivecn, 0, nslot)
        out = wid * BPW + c * GG
        pltpu.sync_copy(gmf_v, gmf_hbm.at[pl.ds(out, GG)])
        pltpu.sync_copy(xcat_v, xcat_hbm.at[pl.ds(out, GG)])
        return carry

    lax.fori_loop(0, NGRP, group_body, 0)
    drain(NH % 2)


@functools.lru_cache(maxsize=1)
def _sc_gather_call():
    mesh = plsc.VectorSubcoreMesh(core_axis_name="c", subcore_axis_name="s")
    return pl.kernel(
        _sc_gather_body,
        out_type=[jax.ShapeDtypeStruct((B, F), jnp.float32),
                  jax.ShapeDtypeStruct((B, 2 * F), jnp.float32)],
        mesh=mesh,
        scratch_types=[
            pltpu.VMEM((IDXROWS, IDXW), jnp.int32),
            pltpu.VMEM((IDXROWS, IDXW), jnp.int32),
            pltpu.VMEM((2, HG, 4, 2, 8, 128), jnp.float32),
            pltpu.VMEM((GG, F), jnp.float32),
            pltpu.VMEM((GG, 2 * F), jnp.float32),
            pltpu.SemaphoreType.DMA,
            pltpu.SemaphoreType.DMA,
        ],
        compiler_params=pltpu.CompilerParams(
            use_tc_tiling_on_sc=True, needs_layout_passes=False),
    )


ROWS_PER_TC_BLOCK = 2048


def _tc_mlp_body(gmf_ref, xcat_ref,
                 w0_ref, b0_ref, w1_ref, b1_ref, w2_ref, b2_ref,
                 wpt_ref, bp_ref, out_ref):
    x = xcat_ref[:]
    h = jnp.dot(x, w0_ref[:], preferred_element_type=jnp.float32) + b0_ref[:]
    h = jnp.maximum(h, 0.0)
    h = jnp.dot(h, w1_ref[:], preferred_element_type=jnp.float32) + b1_ref[:]
    h = jnp.maximum(h, 0.0)
    h = jnp.dot(h, w2_ref[:], preferred_element_type=jnp.float32) + b2_ref[:]
    h = jnp.maximum(h, 0.0)
    con = jnp.concatenate([gmf_ref[:], h], axis=1)
    z = jnp.sum(con * wpt_ref[:], axis=1, keepdims=True) + bp_ref[:]
    out_ref[:] = jax.nn.sigmoid(z)


@functools.lru_cache(maxsize=1)
def _tc_mlp_call():
    nblk = B // ROWS_PER_TC_BLOCK
    full = lambda shape: pl.BlockSpec(shape, lambda i: (0,) * len(shape))
    return pl.pallas_call(
        _tc_mlp_body,
        grid=(nblk,),
        in_specs=[
            pl.BlockSpec((ROWS_PER_TC_BLOCK, F), lambda i: (i, 0)),
            pl.BlockSpec((ROWS_PER_TC_BLOCK, 2 * F), lambda i: (i, 0)),
            full((2 * F, 64)), full((1, 64)),
            full((64, 32)), full((1, 32)),
            full((32, F)), full((1, F)),
            full((1, 2 * F)), full((1, 1)),
        ],
        out_specs=pl.BlockSpec((ROWS_PER_TC_BLOCK, 1), lambda i: (i, 0)),
        out_shape=jax.ShapeDtypeStruct((B, 1), jnp.float32),
    )


def kernel(user_id, item_id, P, Q, U_emb, V_emb, W0, b0, W1, b1, W2, b2, Wp, bp):
    uid = user_id.astype(jnp.int32).reshape(B // IDXW, IDXW)
    iid = item_id.astype(jnp.int32).reshape(B // IDXW, IDXW)
    p3 = P.T.reshape(2, 8, N_ROWS)
    q3 = Q.T.reshape(2, 8, N_ROWS)
    u3 = U_emb.T.reshape(2, 8, N_ROWS)
    v3 = V_emb.T.reshape(2, 8, N_ROWS)
    gmf, xcat = _sc_gather_call()(uid, iid, p3, q3, u3, v3)
    wpt = Wp.reshape(1, 2 * F)
    return _tc_mlp_call()(
        gmf, xcat,
        W0, b0.reshape(1, 64), W1, b1.reshape(1, 32), W2, b2.reshape(1, F),
        wpt, bp.reshape(1, 1),
    )


# speedup vs baseline: 7.2813x; 1.3821x over previous
import functools

import jax
import jax.numpy as jnp
from jax import lax
from jax.experimental import pallas as pl
from jax.experimental.pallas import tpu as pltpu
from jax.experimental.pallas import tpu_sc as plsc

B = 16384
F = 16
N_ROWS = 1000000
NC, NS = 2, 16
NW = NC * NS
BPW = B // NW
IDXW = 128
IDXROWS = BPW // IDXW
GG = 16
NGRP = BPW // GG
HG = 4


def _sc_gather_body(uid_hbm, iid_hbm, p_hbm, q_hbm, u_hbm, v_hbm,
                    gmf_hbm, xcat_hbm,
                    uid_s, iid_s, sbuf, gmf_v, xcat_v, sem0, sem1):
    wid = lax.axis_index("s") * NC + lax.axis_index("c")
    idx_row = wid * IDXROWS
    pltpu.sync_copy(uid_hbm.at[pl.ds(idx_row, IDXROWS)], uid_s)
    pltpu.sync_copy(iid_hbm.at[pl.ds(idx_row, IDXROWS)], iid_s)
    lanes = lax.iota(jnp.int32, 16)
    g_idx = lax.shift_right_logical(lanes, 3)
    f_idx = lax.bitwise_and(lanes, 7)
    sems = (sem0, sem1)
    NH = GG // HG

    def idx_vecs(c):
        r = c // (IDXW // GG)
        col = (c % (IDXW // GG)) * GG
        uvec = uid_s[r, pl.ds(col, GG)]
        ivec = iid_s[r, pl.ds(col, GG)]
        return uvec, ivec

    def fire(uvec, ivec, h, slot):
        sem = sems[slot]
        for i in range(HG):
            j = h * HG + i
            ub = pl.multiple_of(lax.bitwise_and(uvec[j], ~15), 16)
            ib = pl.multiple_of(lax.bitwise_and(ivec[j], ~15), 16)
            pltpu.async_copy(p_hbm.at[:, :, pl.ds(ub, 16)],
                             sbuf.at[slot, i, 0, :, :, pl.ds(0, 16)], sem)
            pltpu.async_copy(q_hbm.at[:, :, pl.ds(ib, 16)],
                             sbuf.at[slot, i, 1, :, :, pl.ds(0, 16)], sem)
            pltpu.async_copy(u_hbm.at[:, :, pl.ds(ub, 16)],
                             sbuf.at[slot, i, 2, :, :, pl.ds(0, 16)], sem)
            pltpu.async_copy(v_hbm.at[:, :, pl.ds(ib, 16)],
                             sbuf.at[slot, i, 3, :, :, pl.ds(0, 16)], sem)

    def drain(slot):
        sem = sems[slot]
        for i in range(HG):
            for t in range(4):
                pltpu.make_async_copy(
                    p_hbm.at[:, :, pl.ds(0, 16)],
                    sbuf.at[slot, i, t, :, :, pl.ds(0, 16)], sem).wait()

    def extract(uvec, ivec, h, slot):
        for i in range(HG):
            j = h * HG + i
            ul = jnp.full((16,), lax.bitwise_and(uvec[j], 15), jnp.int32)
            il = jnp.full((16,), lax.bitwise_and(ivec[j], 15), jnp.int32)
            prow = plsc.load_gather(sbuf.at[slot, i, 0], [g_idx, f_idx, ul])
            qrow = plsc.load_gather(sbuf.at[slot, i, 1], [g_idx, f_idx, il])
            urow = plsc.load_gather(sbuf.at[slot, i, 2], [g_idx, f_idx, ul])
            vrow = plsc.load_gather(sbuf.at[slot, i, 3], [g_idx, f_idx, il])
            gmf_v[j, :] = prow * qrow
            xcat_v[j, pl.ds(0, F)] = urow
            xcat_v[j, pl.ds(F, F)] = vrow

    uvec0, ivec0 = idx_vecs(0)
    fire(uvec0, ivec0, 0, 0)

    def group_body(c, carry):
        uvec, ivec = idx_vecs(c)
        cn = jnp.minimum(c + 1, NGRP - 1)
        uvecn, ivecn = idx_vecs(cn)
        for h in range(NH):
            slot = h % 2
            nslot = (h + 1) % 2
            if h + 1 < NH:
                fire(uvec, ivec, h + 1, nslot)
            drain(slot)
            extract(uvec, ivec, h, slot)
            if h + 1 == NH:
                fire(uvecn, ivecn, 0, nslot)
        out = wid * BPW + c * GG
        pltpu.sync_copy(gmf_v, gmf_hbm.at[pl.ds(out, GG)])
        pltpu.sync_copy(xcat_v, xcat_hbm.at[pl.ds(out, GG)])
        return carry

    lax.fori_loop(0, NGRP, group_body, 0)
    drain(NH % 2)


@functools.lru_cache(maxsize=1)
def _sc_gather_call():
    mesh = plsc.VectorSubcoreMesh(core_axis_name="c", subcore_axis_name="s")
    return pl.kernel(
        _sc_gather_body,
        out_type=[jax.ShapeDtypeStruct((B, F), jnp.float32),
                  jax.ShapeDtypeStruct((B, 2 * F), jnp.float32)],
        mesh=mesh,
        scratch_types=[
            pltpu.VMEM((IDXROWS, IDXW), jnp.int32),
            pltpu.VMEM((IDXROWS, IDXW), jnp.int32),
            pltpu.VMEM((2, HG, 4, 2, 8, 128), jnp.float32),
            pltpu.VMEM((GG, F), jnp.float32),
            pltpu.VMEM((GG, 2 * F), jnp.float32),
            pltpu.SemaphoreType.DMA,
            pltpu.SemaphoreType.DMA,
        ],
        compiler_params=pltpu.CompilerParams(
            use_tc_tiling_on_sc=True, needs_layout_passes=False),
    )


ROWS_PER_TC_BLOCK = 2048


def _tc_mlp_body(gmf_ref, xcat_ref,
                 w0_ref, b0_ref, w1_ref, b1_ref, w2_ref, b2_ref,
                 wpt_ref, bp_ref, out_ref):
    x = xcat_ref[:]
    h = jnp.dot(x, w0_ref[:], preferred_element_type=jnp.float32) + b0_ref[:]
    h = jnp.maximum(h, 0.0)
    h = jnp.dot(h, w1_ref[:], preferred_element_type=jnp.float32) + b1_ref[:]
    h = jnp.maximum(h, 0.0)
    h = jnp.dot(h, w2_ref[:], preferred_element_type=jnp.float32) + b2_ref[:]
    h = jnp.maximum(h, 0.0)
    con = jnp.concatenate([gmf_ref[:], h], axis=1)
    z = jnp.sum(con * wpt_ref[:], axis=1, keepdims=True) + bp_ref[:]
    out_ref[:] = jax.nn.sigmoid(z)


@functools.lru_cache(maxsize=1)
def _tc_mlp_call():
    nblk = B // ROWS_PER_TC_BLOCK
    full = lambda shape: pl.BlockSpec(shape, lambda i: (0,) * len(shape))
    return pl.pallas_call(
        _tc_mlp_body,
        grid=(nblk,),
        in_specs=[
            pl.BlockSpec((ROWS_PER_TC_BLOCK, F), lambda i: (i, 0)),
            pl.BlockSpec((ROWS_PER_TC_BLOCK, 2 * F), lambda i: (i, 0)),
            full((2 * F, 64)), full((1, 64)),
            full((64, 32)), full((1, 32)),
            full((32, F)), full((1, F)),
            full((1, 2 * F)), full((1, 1)),
        ],
        out_specs=pl.BlockSpec((ROWS_PER_TC_BLOCK, 1), lambda i: (i, 0)),
        out_shape=jax.ShapeDtypeStruct((B, 1), jnp.float32),
    )


def kernel(user_id, item_id, P, Q, U_emb, V_emb, W0, b0, W1, b1, W2, b2, Wp, bp):
    uid = user_id.astype(jnp.int32).reshape(B // IDXW, IDXW)
    iid = item_id.astype(jnp.int32).reshape(B // IDXW, IDXW)
    p3 = P.T.reshape(2, 8, N_ROWS)
    q3 = Q.T.reshape(2, 8, N_ROWS)
    u3 = U_emb.T.reshape(2, 8, N_ROWS)
    v3 = V_emb.T.reshape(2, 8, N_ROWS)
    gmf, xcat = _sc_gather_call()(uid, iid, p3, q3, u3, v3)
    wpt = Wp.reshape(1, 2 * F)
    return _tc_mlp_call()(
        gmf, xcat,
        W0, b0.reshape(1, 64), W1, b1.reshape(1, 32), W2, b2.reshape(1, F),
        wpt, bp.reshape(1, 1),
    )


# speedup vs baseline: 7.5832x; 1.0415x over previous
import functools

import jax
import jax.numpy as jnp
from jax import lax
from jax.experimental import pallas as pl
from jax.experimental.pallas import tpu as pltpu
from jax.experimental.pallas import tpu_sc as plsc

B = 16384
F = 16
N_ROWS = 1000000
NC, NS = 2, 16
NW = NC * NS
BPW = B // NW
IDXW = 128
IDXROWS = BPW // IDXW
GG = 16
NGRP = BPW // GG
HG = 8


def _sc_gather_body(uid_hbm, iid_hbm, p_hbm, q_hbm, u_hbm, v_hbm,
                    gmf_hbm, xcat_hbm,
                    uid_s, iid_s, sbuf, gmf_v, xcat_v, sem0, sem1, sem_out):
    wid = lax.axis_index("s") * NC + lax.axis_index("c")
    idx_row = wid * IDXROWS
    pltpu.sync_copy(uid_hbm.at[pl.ds(idx_row, IDXROWS)], uid_s)
    pltpu.sync_copy(iid_hbm.at[pl.ds(idx_row, IDXROWS)], iid_s)
    lanes = lax.iota(jnp.int32, 16)
    g_idx = lax.shift_right_logical(lanes, 3)
    f_idx = lax.bitwise_and(lanes, 7)
    sems = (sem0, sem1)
    tables = (p_hbm, q_hbm, u_hbm, v_hbm)
    NH = GG // HG

    def idx_vecs(c):
        r = c // (IDXW // GG)
        col = (c % (IDXW // GG)) * GG
        uvec = uid_s[r, pl.ds(col, GG)]
        ivec = iid_s[r, pl.ds(col, GG)]
        return uvec, ivec

    def fire(uvec, ivec, h, slot):
        sem = sems[slot]
        for i in range(HG):
            j = h * HG + i
            ub = pl.multiple_of(lax.bitwise_and(uvec[j], ~15), 16)
            ib = pl.multiple_of(lax.bitwise_and(ivec[j], ~15), 16)
            for t, (tab, base) in enumerate(
                    zip(tables, (ub, ib, ub, ib))):
                pltpu.async_copy(
                    tab.at[:, :, pl.ds(base, 16)],
                    sbuf.at[slot, i, :, :, pl.ds(16 * t, 16)], sem)

    def drain(slot):
        sem = sems[slot]
        for _ in range(HG * 4 // 8):
            pltpu.make_async_copy(p_hbm.at[:, :, pl.ds(0, 128)],
                                  sbuf.at[slot, 0], sem).wait()

    def drain_out():
        pltpu.make_async_copy(gmf_v, gmf_hbm.at[pl.ds(0, GG)],
                              sem_out).wait()
        pltpu.make_async_copy(xcat_v, xcat_hbm.at[pl.ds(0, GG)],
                              sem_out).wait()

    def extract(uvec, ivec, h, slot):
        for i in range(HG):
            j = h * HG + i
            ul = lax.bitwise_and(uvec[j], 15)
            il = lax.bitwise_and(ivec[j], 15)
            ref = sbuf.at[slot, i]
            prow = plsc.load_gather(ref, [g_idx, f_idx, jnp.full((16,), ul, jnp.int32)])
            qrow = plsc.load_gather(ref, [g_idx, f_idx, jnp.full((16,), il + 16, jnp.int32)])
            urow = plsc.load_gather(ref, [g_idx, f_idx, jnp.full((16,), ul + 32, jnp.int32)])
            vrow = plsc.load_gather(ref, [g_idx, f_idx, jnp.full((16,), il + 48, jnp.int32)])
            gmf_v[j, :] = prow * qrow
            xcat_v[j, pl.ds(0, F)] = urow
            xcat_v[j, pl.ds(F, F)] = vrow

    uvec0, ivec0 = idx_vecs(0)
    fire(uvec0, ivec0, 0, 0)

    def group_body(c, carry):
        uvec, ivec = idx_vecs(c)
        cn = jnp.minimum(c + 1, NGRP - 1)
        uvecn, ivecn = idx_vecs(cn)

        @pl.when(c > 0)
        def _():
            drain_out()

        for h in range(NH):
            slot = h % 2
            nslot = (h + 1) % 2
            if h + 1 < NH:
                fire(uvec, ivec, h + 1, nslot)
            drain(slot)
            extract(uvec, ivec, h, slot)
            if h + 1 == NH:
                fire(uvecn, ivecn, 0, nslot)
        out = wid * BPW + c * GG
        pltpu.async_copy(gmf_v, gmf_hbm.at[pl.ds(out, GG)], sem_out)
        pltpu.async_copy(xcat_v, xcat_hbm.at[pl.ds(out, GG)], sem_out)
        return carry

    lax.fori_loop(0, NGRP, group_body, 0)
    drain_out()
    drain(NH % 2)


@functools.lru_cache(maxsize=1)
def _sc_gather_call():
    mesh = plsc.VectorSubcoreMesh(core_axis_name="c", subcore_axis_name="s")
    return pl.kernel(
        _sc_gather_body,
        out_type=[jax.ShapeDtypeStruct((B, F), jnp.float32),
                  jax.ShapeDtypeStruct((B, 2 * F), jnp.float32)],
        mesh=mesh,
        scratch_types=[
            pltpu.VMEM((IDXROWS, IDXW), jnp.int32),
            pltpu.VMEM((IDXROWS, IDXW), jnp.int32),
            pltpu.VMEM((2, HG, 2, 8, 128), jnp.float32),
            pltpu.VMEM((GG, F), jnp.float32),
            pltpu.VMEM((GG, 2 * F), jnp.float32),
            pltpu.SemaphoreType.DMA,
            pltpu.SemaphoreType.DMA,
            pltpu.SemaphoreType.DMA,
        ],
        compiler_params=pltpu.CompilerParams(
            use_tc_tiling_on_sc=True, needs_layout_passes=False),
    )


ROWS_PER_TC_BLOCK = 2048


def _tc_mlp_body(gmf_ref, xcat_ref,
                 w0_ref, b0_ref, w1_ref, b1_ref, w2_ref, b2_ref,
                 wpt_ref, bp_ref, out_ref):
    x = xcat_ref[:]
    h = jnp.dot(x, w0_ref[:], preferred_element_type=jnp.float32) + b0_ref[:]
    h = jnp.maximum(h, 0.0)
    h = jnp.dot(h, w1_ref[:], preferred_element_type=jnp.float32) + b1_ref[:]
    h = jnp.maximum(h, 0.0)
    h = jnp.dot(h, w2_ref[:], preferred_element_type=jnp.float32) + b2_ref[:]
    h = jnp.maximum(h, 0.0)
    con = jnp.concatenate([gmf_ref[:], h], axis=1)
    z = jnp.sum(con * wpt_ref[:], axis=1, keepdims=True) + bp_ref[:]
    out_ref[:] = jax.nn.sigmoid(z)


@functools.lru_cache(maxsize=1)
def _tc_mlp_call():
    nblk = B // ROWS_PER_TC_BLOCK
    full = lambda shape: pl.BlockSpec(shape, lambda i: (0,) * len(shape))
    return pl.pallas_call(
        _tc_mlp_body,
        grid=(nblk,),
        in_specs=[
            pl.BlockSpec((ROWS_PER_TC_BLOCK, F), lambda i: (i, 0)),
            pl.BlockSpec((ROWS_PER_TC_BLOCK, 2 * F), lambda i: (i, 0)),
            full((2 * F, 64)), full((1, 64)),
            full((64, 32)), full((1, 32)),
            full((32, F)), full((1, F)),
            full((1, 2 * F)), full((1, 1)),
        ],
        out_specs=pl.BlockSpec((ROWS_PER_TC_BLOCK, 1), lambda i: (i, 0)),
        out_shape=jax.ShapeDtypeStruct((B, 1), jnp.float32),
    )


def kernel(user_id, item_id, P, Q, U_emb, V_emb, W0, b0, W1, b1, W2, b2, Wp, bp):
    uid = user_id.astype(jnp.int32).reshape(B // IDXW, IDXW)
    iid = item_id.astype(jnp.int32).reshape(B // IDXW, IDXW)
    p3 = P.T.reshape(2, 8, N_ROWS)
    q3 = Q.T.reshape(2, 8, N_ROWS)
    u3 = U_emb.T.reshape(2, 8, N_ROWS)
    v3 = V_emb.T.reshape(2, 8, N_ROWS)
    gmf, xcat = _sc_gather_call()(uid, iid, p3, q3, u3, v3)
    wpt = Wp.reshape(1, 2 * F)
    return _tc_mlp_call()(
        gmf, xcat,
        W0, b0.reshape(1, 64), W1, b1.reshape(1, 32), W2, b2.reshape(1, F),
        wpt, bp.reshape(1, 1),
    )


# speedup vs baseline: 8.6452x; 1.1400x over previous
import functools

import jax
import jax.numpy as jnp
from jax import lax
from jax.experimental import pallas as pl
from jax.experimental.pallas import tpu as pltpu
from jax.experimental.pallas import tpu_sc as plsc

B = 16384
F = 16
N_ROWS = 1000000
NC, NS = 2, 16
NW = NC * NS
BPW = B // NW
IDXW = 128
IDXROWS = BPW // IDXW
GG = 16
NGRP = BPW // GG
HG = 4


NHG = BPW // HG
CPG = IDXW // GG
UNROLL = 4
NB = 4


def _sc_gather_body(uid_hbm, iid_hbm, p_hbm, q_hbm, u_hbm, v_hbm,
                    gmf_hbm, xcat_hbm,
                    uid_s, iid_s, sbuf, gmf_a, xcat_a,
                    sem0, sem1, sem2, sem3, sem_oa, sem_ob):
    wid = lax.axis_index("s") * NC + lax.axis_index("c")
    idx_row = wid * IDXROWS
    pltpu.sync_copy(uid_hbm.at[pl.ds(idx_row, IDXROWS)], uid_s)
    pltpu.sync_copy(iid_hbm.at[pl.ds(idx_row, IDXROWS)], iid_s)
    lanes = lax.iota(jnp.int32, 16)
    g_idx = lax.shift_right_logical(lanes, 3)
    f_idx = lax.bitwise_and(lanes, 7)
    sems = (sem0, sem1, sem2, sem3)
    tables = (p_hbm, q_hbm, u_hbm, v_hbm)

    def hg_vecs(hg):
        g = hg // (GG // HG)
        r = g // CPG
        col = pl.multiple_of((g % CPG) * GG, 16)
        return uid_s[r, pl.ds(col, GG)], iid_s[r, pl.ds(col, GG)]

    def fire(hg, parity, slot):
        uvec, ivec = hg_vecs(hg)
        sem = sems[slot]
        for i in range(HG):
            j = parity * HG + i
            ub = pl.multiple_of(lax.bitwise_and(uvec[j], ~15), 16)
            ib = pl.multiple_of(lax.bitwise_and(ivec[j], ~15), 16)
            for t, (tab, base) in enumerate(zip(tables, (ub, ib, ub, ib))):
                pltpu.async_copy(
                    tab.at[:, :, pl.ds(base, 16)],
                    sbuf.at[slot, i, :, :, pl.ds(16 * t, 16)], sem)

    def drain(slot):
        sem = sems[slot]
        for _ in range(HG * 4 // 8):
            pltpu.make_async_copy(p_hbm.at[:, :, pl.ds(0, 128)],
                                  sbuf.at[slot, 0], sem).wait()

    def drain_out(sem_o):
        pltpu.make_async_copy(gmf_a.at[pl.ds(0, 8)],
                              gmf_hbm.at[pl.ds(0, 8)], sem_o).wait()
        pltpu.make_async_copy(xcat_a.at[pl.ds(0, 8)],
                              xcat_hbm.at[pl.ds(0, 8)], sem_o).wait()

    def extract(hg, quarter, slot):
        uvec, ivec = hg_vecs(hg)
        gmf_v, xcat_v = gmf_a, xcat_a
        for i in range(HG):
            j = quarter * HG + i
            ul = lax.bitwise_and(uvec[j], 15)
            il = lax.bitwise_and(ivec[j], 15)
            ref = sbuf.at[slot, i]
            prow = plsc.load_gather(
                ref, [g_idx, f_idx, jnp.full((16,), ul, jnp.int32)])
            qrow = plsc.load_gather(
                ref, [g_idx, f_idx, jnp.full((16,), il + 16, jnp.int32)])
            urow = plsc.load_gather(
                ref, [g_idx, f_idx, jnp.full((16,), ul + 32, jnp.int32)])
            vrow = plsc.load_gather(
                ref, [g_idx, f_idx, jnp.full((16,), il + 48, jnp.int32)])
            gmf_v[j, :] = prow * qrow
            xcat_v[j, pl.ds(0, F)] = urow
            xcat_v[j, pl.ds(F, F)] = vrow

    fire(0, 0, 0)
    fire(1, 1, 1)
    fire(2, 2, 2)

    def iter_body(it, carry):
        hg0 = it * UNROLL
        out = wid * BPW + it * GG
        for k in range(UNROLL):
            hg = hg0 + k
            slot = k % NB
            if k == 0:
                @pl.when(it > 0)
                def _():
                    drain_out(sem_oa)
            if k == 2:
                @pl.when(it > 0)
                def _():
                    drain_out(sem_ob)
            drain(slot)
            extract(hg, k, slot)
            if k == 1:
                pltpu.async_copy(gmf_a.at[pl.ds(0, 8)],
                                 gmf_hbm.at[pl.ds(out, 8)], sem_oa)
                pltpu.async_copy(xcat_a.at[pl.ds(0, 8)],
                                 xcat_hbm.at[pl.ds(out, 8)], sem_oa)
            if k == 3:
                pltpu.async_copy(gmf_a.at[pl.ds(8, 8)],
                                 gmf_hbm.at[pl.ds(out + 8, 8)], sem_ob)
                pltpu.async_copy(xcat_a.at[pl.ds(8, 8)],
                                 xcat_hbm.at[pl.ds(out + 8, 8)], sem_ob)
            nhg = jnp.minimum(hg + 3, NHG - 1)
            fire(nhg, (k + 3) % UNROLL, (k + 3) % NB)
        return carry

    lax.fori_loop(0, NHG // UNROLL, iter_body, 0)
    drain_out(sem_oa)
    drain_out(sem_ob)
    drain(0)
    drain(1)
    drain(2)


@functools.lru_cache(maxsize=1)
def _sc_gather_call():
    mesh = plsc.VectorSubcoreMesh(core_axis_name="c", subcore_axis_name="s")
    return pl.kernel(
        _sc_gather_body,
        out_type=[jax.ShapeDtypeStruct((B, F), jnp.float32),
                  jax.ShapeDtypeStruct((B, 2 * F), jnp.float32)],
        mesh=mesh,
        scratch_types=[
            pltpu.VMEM((IDXROWS, IDXW), jnp.int32),
            pltpu.VMEM((IDXROWS, IDXW), jnp.int32),
            pltpu.VMEM((NB, HG, 2, 8, 128), jnp.float32),
            pltpu.VMEM((GG, F), jnp.float32),
            pltpu.VMEM((GG, 2 * F), jnp.float32),
            pltpu.SemaphoreType.DMA,
            pltpu.SemaphoreType.DMA,
            pltpu.SemaphoreType.DMA,
            pltpu.SemaphoreType.DMA,
            pltpu.SemaphoreType.DMA,
            pltpu.SemaphoreType.DMA,
        ],
        compiler_params=pltpu.CompilerParams(
            use_tc_tiling_on_sc=True, needs_layout_passes=False),
    )


ROWS_PER_TC_BLOCK = 2048


def _tc_mlp_body(gmf_ref, xcat_ref,
                 w0_ref, b0_ref, w1_ref, b1_ref, w2_ref, b2_ref,
                 wpt_ref, bp_ref, out_ref):
    x = xcat_ref[:]
    h = jnp.dot(x, w0_ref[:], preferred_element_type=jnp.float32) + b0_ref[:]
    h = jnp.maximum(h, 0.0)
    h = jnp.dot(h, w1_ref[:], preferred_element_type=jnp.float32) + b1_ref[:]
    h = jnp.maximum(h, 0.0)
    h = jnp.dot(h, w2_ref[:], preferred_element_type=jnp.float32) + b2_ref[:]
    h = jnp.maximum(h, 0.0)
    con = jnp.concatenate([gmf_ref[:], h], axis=1)
    z = jnp.sum(con * wpt_ref[:], axis=1, keepdims=True) + bp_ref[:]
    out_ref[:] = jax.nn.sigmoid(z)


@functools.lru_cache(maxsize=1)
def _tc_mlp_call():
    nblk = B // ROWS_PER_TC_BLOCK
    full = lambda shape: pl.BlockSpec(shape, lambda i: (0,) * len(shape))
    return pl.pallas_call(
        _tc_mlp_body,
        grid=(nblk,),
        in_specs=[
            pl.BlockSpec((ROWS_PER_TC_BLOCK, F), lambda i: (i, 0)),
            pl.BlockSpec((ROWS_PER_TC_BLOCK, 2 * F), lambda i: (i, 0)),
            full((2 * F, 64)), full((1, 64)),
            full((64, 32)), full((1, 32)),
            full((32, F)), full((1, F)),
            full((1, 2 * F)), full((1, 1)),
        ],
        out_specs=pl.BlockSpec((ROWS_PER_TC_BLOCK, 1), lambda i: (i, 0)),
        out_shape=jax.ShapeDtypeStruct((B, 1), jnp.float32),
    )


def kernel(user_id, item_id, P, Q, U_emb, V_emb, W0, b0, W1, b1, W2, b2, Wp, bp):
    uid = user_id.astype(jnp.int32).reshape(B // IDXW, IDXW)
    iid = item_id.astype(jnp.int32).reshape(B // IDXW, IDXW)
    p3 = P.T.reshape(2, 8, N_ROWS)
    q3 = Q.T.reshape(2, 8, N_ROWS)
    u3 = U_emb.T.reshape(2, 8, N_ROWS)
    v3 = V_emb.T.reshape(2, 8, N_ROWS)
    gmf, xcat = _sc_gather_call()(uid, iid, p3, q3, u3, v3)
    wpt = Wp.reshape(1, 2 * F)
    return _tc_mlp_call()(
        gmf, xcat,
        W0, b0.reshape(1, 64), W1, b1.reshape(1, 32), W2, b2.reshape(1, F),
        wpt, bp.reshape(1, 1),
    )
